# baseline (speedup 1.0000x reference)
from itertools import permutations

import jax
import jax.numpy as jnp
import numpy as np
from jax.experimental import pallas as pl
from jax.experimental.pallas import tpu as pltpu

B = 4
L = 512
NH = 12
H = 768
E = 16
DIS = 20
TYPE = 20
TAG = 7
REL = 97
BLK = 64
P = E * (E - 1)
G = H // BLK
HIN = H * 2 + DIS + TYPE

_HTS = np.array(list(permutations(range(E), 2)), dtype=np.int32)
_G0_NP = np.zeros((P, E), np.float32)
_G0_NP[np.arange(P), _HTS[:, 0]] = 1.0
_G1_NP = np.zeros((P, E), np.float32)
_G1_NP[np.arange(P), _HTS[:, 1]] = 1.0
_KRON_NP = np.zeros((BLK, BLK * BLK), np.float32)
for _i in range(BLK):
    _KRON_NP[_i, _i * BLK:(_i + 1) * BLK] = 1.0


def _fused_kernel(head_col, tail_col, et_col, mask, att, hs, g0, g1,
                  wh, bh, wt, bt, type_emb, dis_emb, w3, cbias, kron,
                  pred_out, ah_scr, at_scr, acc_scr):
    gi = pl.program_id(1)

    @pl.when(gi == 0)
    def _stage1():
        lane_iota = jax.lax.broadcasted_iota(jnp.int32, (E, L), 1)
        hc = head_col[0]
        tc = tail_col[0]
        s = 0.5 * ((lane_iota == hc).astype(jnp.float32)
                   + (lane_iota == tc).astype(jnp.float32))
        ea = jnp.dot(s, att[0, 0], preferred_element_type=jnp.float32)
        for h in range(1, NH):
            ea += jnp.dot(s, att[0, h], preferred_element_type=jnp.float32)
        hs2 = hs[0]
        ef = jnp.dot(s, hs2, preferred_element_type=jnp.float32)
        g0v = g0[...]
        g1v = g1[...]
        pa = (jnp.dot(g0v, ea, preferred_element_type=jnp.float32)
              * jnp.dot(g1v, ea, preferred_element_type=jnp.float32)
              * mask[0])
        pa = pa / (jnp.sum(pa, axis=1, keepdims=True) + 1e-20)
        info = jnp.dot(pa, hs2, preferred_element_type=jnp.float32)
        hf = jnp.dot(g0v, ef, preferred_element_type=jnp.float32)
        tf = jnp.dot(g1v, ef, preferred_element_type=jnp.float32)
        t_iota = jax.lax.broadcasted_iota(jnp.int32, (E, TAG), 1)
        t_oh = (t_iota == et_col[0]).astype(jnp.float32)
        tfeat = jnp.dot(t_oh, type_emb[...],
                        preferred_element_type=jnp.float32)
        htype = jnp.dot(g0v, tfeat, preferred_element_type=jnp.float32)
        ttype = jnp.dot(g1v, tfeat, preferred_element_type=jnp.float32)
        x = jnp.concatenate([tc, hc], axis=1).astype(jnp.float32)
        y = jnp.dot(g0v, x, preferred_element_type=jnp.float32)
        z = jnp.dot(g1v, x, preferred_element_type=jnp.float32)
        d = jnp.abs(y[:, 0:1] - z[:, 1:2])
        bucket = jnp.zeros_like(d)
        for thr in (2., 4., 8., 16., 32., 64., 128., 256., 512.):
            bucket += (d >= thr).astype(jnp.float32)
        d_iota = jax.lax.broadcasted_iota(jnp.int32, (P, DIS), 1)
        d_oh = (d_iota == bucket.astype(jnp.int32)).astype(jnp.float32)
        dfeat = jnp.dot(d_oh, dis_emb[...],
                        preferred_element_type=jnp.float32)
        lh = jnp.concatenate([hf, info], axis=1)
        lt = jnp.concatenate([tf, info], axis=1)
        sh = jnp.concatenate([htype, dfeat], axis=1)
        st = jnp.concatenate([ttype, dfeat], axis=1)
        ah = jnp.tanh(
            jnp.dot(lh, wh[0:2 * H], preferred_element_type=jnp.float32)
            + jnp.dot(sh, wh[2 * H:], preferred_element_type=jnp.float32)
            + bh[...])
        at = jnp.tanh(
            jnp.dot(lt, wt[0:2 * H], preferred_element_type=jnp.float32)
            + jnp.dot(st, wt[2 * H:], preferred_element_type=jnp.float32)
            + bt[...])
        for g in range(G):
            ah_scr[g] = ah[:, g * BLK:(g + 1) * BLK]
            at_scr[g] = at[:, g * BLK:(g + 1) * BLK]

    a1g = ah_scr[gi]
    a2g = at_scr[gi]
    outer = _build_outer(a1g, a2g, kron[...])
    contrib = jnp.dot(outer, w3[0], preferred_element_type=jnp.float32)

    @pl.when(gi == 0)
    def _init():
        acc_scr[...] = contrib

    @pl.when(gi > 0)
    def _accum():
        acc_scr[...] += contrib

    @pl.when(gi == G - 1)
    def _emit():
        pred_out[0] = acc_scr[...] + cbias[...]


def _build_outer(a1v, a2v, kron):
    a1rep = jnp.dot(a1v, kron, preferred_element_type=jnp.float32)
    a2_128 = jnp.concatenate([a2v, a2v], axis=1)
    a2til = jnp.concatenate([a2_128] * (BLK // 2), axis=1)
    return a1rep * a2til


def kernel(hidden_state, attention, head, tail, entity_type, attention_mask,
           h_dense_w, h_dense_b, t_dense_w, t_dense_b, dis_emb, type_emb,
           cls_w, cls_b):
    f32 = jnp.float32
    head_col = head.astype(jnp.int32).reshape(B, E, 1)
    tail_col = tail.astype(jnp.int32).reshape(B, E, 1)
    et_col = entity_type.astype(jnp.int32).reshape(B, E, 1)
    mask3 = attention_mask.reshape(B, 1, L)
    g0 = jnp.asarray(_G0_NP)
    g1 = jnp.asarray(_G1_NP)
    bh = h_dense_b.reshape(1, H)
    bt = t_dense_b.reshape(1, H)

    w3 = cls_w.reshape(G, BLK * BLK, REL)
    cbias = cls_b.reshape(1, REL)
    kron = jnp.asarray(_KRON_NP)

    const = lambda shape: pl.BlockSpec(shape, lambda b, g: tuple(0 for _ in shape))
    per_b = lambda shape: pl.BlockSpec(shape, lambda b, g: (b,) + tuple(0 for _ in shape[1:]))
    pred4 = pl.pallas_call(
        _fused_kernel,
        grid=(B, G),
        in_specs=[
            per_b((1, E, 1)),
            per_b((1, E, 1)),
            per_b((1, E, 1)),
            per_b((1, 1, L)),
            per_b((1, NH, L, L)),
            per_b((1, L, H)),
            const((P, E)),
            const((P, E)),
            const((HIN, H)),
            const((1, H)),
            const((HIN, H)),
            const((1, H)),
            const((TAG, TYPE)),
            const((DIS, DIS)),
            pl.BlockSpec((1, BLK * BLK, REL), lambda b, g: (g, 0, 0)),
            const((1, REL)),
            const((BLK, BLK * BLK)),
        ],
        out_specs=per_b((1, P, REL)),
        out_shape=jax.ShapeDtypeStruct((B, P, REL), f32),
        scratch_shapes=[
            pltpu.VMEM((G, P, BLK), f32),
            pltpu.VMEM((G, P, BLK), f32),
            pltpu.VMEM((P, REL), f32),
        ],
    )(head_col, tail_col, et_col, mask3, attention, hidden_state, g0, g1,
      h_dense_w, bh, t_dense_w, bt, type_emb, dis_emb, w3, cbias, kron)
    return pred4.reshape(B * P, REL)


# speedup vs baseline: 1.2031x; 1.2031x over previous
from itertools import permutations

import jax
import jax.numpy as jnp
import numpy as np
from jax.experimental import pallas as pl
from jax.experimental.pallas import tpu as pltpu

B = 4
L = 512
NH = 12
H = 768
E = 16
DIS = 20
TYPE = 20
TAG = 7
REL = 97
BLK = 64
P = E * (E - 1)
G = H // BLK
HIN = H * 2 + DIS + TYPE

_HTS = np.array(list(permutations(range(E), 2)), dtype=np.int32)
_G0_NP = np.zeros((P, E), np.float32)
_G0_NP[np.arange(P), _HTS[:, 0]] = 1.0
_G1_NP = np.zeros((P, E), np.float32)
_G1_NP[np.arange(P), _HTS[:, 1]] = 1.0
_KRON_NP = np.zeros((BLK, BLK * BLK), np.float32)
for _i in range(BLK):
    _KRON_NP[_i, _i * BLK:(_i + 1) * BLK] = 1.0


def _col(row_f32):
    eye = (jax.lax.broadcasted_iota(jnp.int32, (E, E), 0)
           == jax.lax.broadcasted_iota(jnp.int32, (E, E), 1))
    diag = jnp.where(eye, jnp.broadcast_to(row_f32, (E, E)), 0.0)
    return jnp.sum(diag, axis=1, keepdims=True)


def _stage1_kernel(head_a, tail_a, et_a, mask_a, att, hs, g0, g1,
                   wh, bh, wt, bt, type_emb, dis_emb,
                   ah_out, at_out):
    b = pl.program_id(0)
    bmask = jax.lax.broadcasted_iota(jnp.int32, (B, E), 0) == b
    hrow = jnp.sum(jnp.where(bmask, head_a[...], 0), axis=0, keepdims=True)
    trow = jnp.sum(jnp.where(bmask, tail_a[...], 0), axis=0, keepdims=True)
    erow = jnp.sum(jnp.where(bmask, et_a[...], 0), axis=0, keepdims=True)
    hc_f = _col(hrow.astype(jnp.float32))
    tc_f = _col(trow.astype(jnp.float32))
    hc = hc_f.astype(jnp.int32)
    tc = tc_f.astype(jnp.int32)
    et_col = _col(erow.astype(jnp.float32)).astype(jnp.int32)
    bmask_l = jax.lax.broadcasted_iota(jnp.int32, (B, L), 0) == b
    mask = jnp.sum(jnp.where(bmask_l, mask_a[...], 0.0), axis=0, keepdims=True)

    lane_iota = jax.lax.broadcasted_iota(jnp.int32, (E, L), 1)
    s = 0.5 * ((lane_iota == hc).astype(jnp.float32)
               + (lane_iota == tc).astype(jnp.float32))
    ea = jnp.dot(s, att[0, 0], preferred_element_type=jnp.float32)
    for h in range(1, NH):
        ea += jnp.dot(s, att[0, h], preferred_element_type=jnp.float32)
    if True:
        hs2 = hs[0]
        ef = jnp.dot(s, hs2, preferred_element_type=jnp.float32)
        g0v = g0[...]
        g1v = g1[...]
        pa = (jnp.dot(g0v, ea, preferred_element_type=jnp.float32)
              * jnp.dot(g1v, ea, preferred_element_type=jnp.float32)
              * mask)
        pa = pa / (jnp.sum(pa, axis=1, keepdims=True) + 1e-20)
        info = jnp.dot(pa, hs2, preferred_element_type=jnp.float32)
        hf = jnp.dot(g0v, ef, preferred_element_type=jnp.float32)
        tf = jnp.dot(g1v, ef, preferred_element_type=jnp.float32)
        t_iota = jax.lax.broadcasted_iota(jnp.int32, (E, TAG), 1)
        t_oh = (t_iota == et_col).astype(jnp.float32)
        tfeat = jnp.dot(t_oh, type_emb[...],
                        preferred_element_type=jnp.float32)
        htype = jnp.dot(g0v, tfeat, preferred_element_type=jnp.float32)
        ttype = jnp.dot(g1v, tfeat, preferred_element_type=jnp.float32)
        x = jnp.concatenate([tc_f, hc_f], axis=1)
        y = jnp.dot(g0v, x, preferred_element_type=jnp.float32)
        z = jnp.dot(g1v, x, preferred_element_type=jnp.float32)
        d = jnp.abs(y[:, 0:1] - z[:, 1:2])
        bucket = jnp.zeros_like(d)
        for thr in (2., 4., 8., 16., 32., 64., 128., 256., 512.):
            bucket += (d >= thr).astype(jnp.float32)
        d_iota = jax.lax.broadcasted_iota(jnp.int32, (P, DIS), 1)
        d_oh = (d_iota == bucket.astype(jnp.int32)).astype(jnp.float32)
        dfeat = jnp.dot(d_oh, dis_emb[...],
                        preferred_element_type=jnp.float32)
        lh = jnp.concatenate([hf, info], axis=1)
        lt = jnp.concatenate([tf, info], axis=1)
        sh = jnp.concatenate([htype, dfeat], axis=1)
        st = jnp.concatenate([ttype, dfeat], axis=1)
        ah = jnp.tanh(
            jnp.dot(lh, wh[0:2 * H], preferred_element_type=jnp.float32)
            + jnp.dot(sh, wh[2 * H:], preferred_element_type=jnp.float32)
            + bh[...])
        at = jnp.tanh(
            jnp.dot(lt, wt[0:2 * H], preferred_element_type=jnp.float32)
            + jnp.dot(st, wt[2 * H:], preferred_element_type=jnp.float32)
            + bt[...])
        for g in range(G):
            ah_out[g, 0] = ah[:, g * BLK:(g + 1) * BLK]
            at_out[g, 0] = at[:, g * BLK:(g + 1) * BLK]


def _build_outer(a1v, a2v, kron):
    a1rep = jnp.dot(a1v, kron, preferred_element_type=jnp.float32)
    a2_128 = jnp.concatenate([a2v, a2v], axis=1)
    a2til = jnp.concatenate([a2_128] * (BLK // 2), axis=1)
    return a1rep * a2til


def _stage2_kernel(a1, a2, w, bias, kron, out_ref):
    g = pl.program_id(0)
    a1v = a1[0]
    a2v = a2[0]
    outer = _build_outer(a1v, a2v, kron[...])
    contrib = jnp.dot(outer, w[0], preferred_element_type=jnp.float32)

    @pl.when(g == 0)
    def _():
        out_ref[...] = contrib + bias[...]

    @pl.when(g > 0)
    def _():
        out_ref[...] += contrib


def kernel(hidden_state, attention, head, tail, entity_type, attention_mask,
           h_dense_w, h_dense_b, t_dense_w, t_dense_b, dis_emb, type_emb,
           cls_w, cls_b):
    f32 = jnp.float32
    g0 = jnp.asarray(_G0_NP)
    g1 = jnp.asarray(_G1_NP)
    bh = h_dense_b.reshape(1, H)
    bt = t_dense_b.reshape(1, H)

    const = lambda shape: pl.BlockSpec(shape, lambda b: tuple(0 for _ in shape))
    per_b = lambda shape: pl.BlockSpec(shape, lambda b: (b,) + tuple(0 for _ in shape[1:]))
    ah, at = pl.pallas_call(
        _stage1_kernel,
        grid=(B,),
        in_specs=[
            const((B, E)),
            const((B, E)),
            const((B, E)),
            const((B, L)),
            per_b((1, NH, L, L)),
            per_b((1, L, H)),
            const((P, E)),
            const((P, E)),
            const((HIN, H)),
            const((1, H)),
            const((HIN, H)),
            const((1, H)),
            const((TAG, TYPE)),
            const((DIS, DIS)),
        ],
        out_specs=[pl.BlockSpec((G, 1, P, BLK), lambda b: (0, b, 0, 0))] * 2,
        out_shape=[jax.ShapeDtypeStruct((G, B, P, BLK), f32)] * 2,
    )(head, tail, entity_type, attention_mask, attention, hidden_state, g0, g1,
      h_dense_w, bh, t_dense_w, bt, type_emb, dis_emb)

    a1 = ah.reshape(G, B * P, BLK)
    a2 = at.reshape(G, B * P, BLK)
    w3 = cls_w.reshape(G, BLK * BLK, REL)
    bias = cls_b.reshape(1, REL)
    kron = jnp.asarray(_KRON_NP)

    pred = pl.pallas_call(
        _stage2_kernel,
        grid=(G,),
        in_specs=[
            pl.BlockSpec((1, B * P, BLK), lambda g: (g, 0, 0)),
            pl.BlockSpec((1, B * P, BLK), lambda g: (g, 0, 0)),
            pl.BlockSpec((1, BLK * BLK, REL), lambda g: (g, 0, 0)),
            pl.BlockSpec((1, REL), lambda g: (0, 0)),
            pl.BlockSpec((BLK, BLK * BLK), lambda g: (0, 0)),
        ],
        out_specs=pl.BlockSpec((B * P, REL), lambda g: (0, 0)),
        out_shape=jax.ShapeDtypeStruct((B * P, REL), f32),
    )(a1, a2, w3, bias, kron)
    return pred


# speedup vs baseline: 1.2032x; 1.0000x over previous
from itertools import permutations

import jax
import jax.numpy as jnp
import numpy as np
from jax.experimental import pallas as pl
from jax.experimental.pallas import tpu as pltpu

B = 4
L = 512
NH = 12
H = 768
E = 16
DIS = 20
TYPE = 20
TAG = 7
REL = 97
BLK = 64
P = E * (E - 1)
G = H // BLK
HIN = H * 2 + DIS + TYPE

_HTS = np.array(list(permutations(range(E), 2)), dtype=np.int32)
_G0_NP = np.zeros((P, E), np.float32)
_G0_NP[np.arange(P), _HTS[:, 0]] = 1.0
_G1_NP = np.zeros((P, E), np.float32)
_G1_NP[np.arange(P), _HTS[:, 1]] = 1.0
_KRON_NP = np.zeros((BLK, BLK * BLK), np.float32)
for _i in range(BLK):
    _KRON_NP[_i, _i * BLK:(_i + 1) * BLK] = 1.0


def _col(row_f32):
    eye = (jax.lax.broadcasted_iota(jnp.int32, (E, E), 0)
           == jax.lax.broadcasted_iota(jnp.int32, (E, E), 1))
    diag = jnp.where(eye, jnp.broadcast_to(row_f32, (E, E)), 0.0)
    return jnp.sum(diag, axis=1, keepdims=True)


def _stage1_kernel(head_a, tail_a, et_a, mask_a, att, hs, g0, g1,
                   wh, bh, wt, bt, type_emb, dis_emb,
                   ah_out, at_out):
    b = pl.program_id(0)
    bmask = jax.lax.broadcasted_iota(jnp.int32, (B, E), 0) == b
    hrow = jnp.sum(jnp.where(bmask, head_a[...], 0), axis=0, keepdims=True)
    trow = jnp.sum(jnp.where(bmask, tail_a[...], 0), axis=0, keepdims=True)
    erow = jnp.sum(jnp.where(bmask, et_a[...], 0), axis=0, keepdims=True)
    hc_f = _col(hrow.astype(jnp.float32))
    tc_f = _col(trow.astype(jnp.float32))
    hc = hc_f.astype(jnp.int32)
    tc = tc_f.astype(jnp.int32)
    et_col = _col(erow.astype(jnp.float32)).astype(jnp.int32)
    bmask_l = jax.lax.broadcasted_iota(jnp.int32, (B, L), 0) == b
    mask = jnp.sum(jnp.where(bmask_l, mask_a[...], 0.0), axis=0, keepdims=True)

    lane_iota = jax.lax.broadcasted_iota(jnp.int32, (E, L), 1)
    s = 0.5 * ((lane_iota == hc).astype(jnp.float32)
               + (lane_iota == tc).astype(jnp.float32))
    ea = jnp.dot(s, att[0, 0], preferred_element_type=jnp.float32)
    for h in range(1, NH):
        ea += jnp.dot(s, att[0, h], preferred_element_type=jnp.float32)
    if True:
        hs2 = hs[0]
        ef = jnp.dot(s, hs2, preferred_element_type=jnp.float32)
        g0v = g0[...]
        g1v = g1[...]
        pa = (jnp.dot(g0v, ea, preferred_element_type=jnp.float32)
              * jnp.dot(g1v, ea, preferred_element_type=jnp.float32)
              * mask)
        pa = pa / (jnp.sum(pa, axis=1, keepdims=True) + 1e-20)
        info = jnp.dot(pa, hs2, preferred_element_type=jnp.float32)
        hf = jnp.dot(g0v, ef, preferred_element_type=jnp.float32)
        tf = jnp.dot(g1v, ef, preferred_element_type=jnp.float32)
        t_iota = jax.lax.broadcasted_iota(jnp.int32, (E, TAG), 1)
        t_oh = (t_iota == et_col).astype(jnp.float32)
        tfeat = jnp.dot(t_oh, type_emb[...],
                        preferred_element_type=jnp.float32)
        htype = jnp.dot(g0v, tfeat, preferred_element_type=jnp.float32)
        ttype = jnp.dot(g1v, tfeat, preferred_element_type=jnp.float32)
        x = jnp.concatenate([tc_f, hc_f], axis=1)
        y = jnp.dot(g0v, x, preferred_element_type=jnp.float32)
        z = jnp.dot(g1v, x, preferred_element_type=jnp.float32)
        d = jnp.abs(y[:, 0:1] - z[:, 1:2])
        bucket = jnp.zeros_like(d)
        for thr in (2., 4., 8., 16., 32., 64., 128., 256., 512.):
            bucket += (d >= thr).astype(jnp.float32)
        d_iota = jax.lax.broadcasted_iota(jnp.int32, (P, DIS), 1)
        d_oh = (d_iota == bucket.astype(jnp.int32)).astype(jnp.float32)
        dfeat = jnp.dot(d_oh, dis_emb[...],
                        preferred_element_type=jnp.float32)
        lh = jnp.concatenate([hf, info], axis=1)
        lt = jnp.concatenate([tf, info], axis=1)
        sh = jnp.concatenate([htype, dfeat], axis=1)
        st = jnp.concatenate([ttype, dfeat], axis=1)
        ah = jnp.tanh(
            jnp.dot(lh, wh[0:2 * H], preferred_element_type=jnp.float32)
            + jnp.dot(sh, wh[2 * H:], preferred_element_type=jnp.float32)
            + bh[...])
        at = jnp.tanh(
            jnp.dot(lt, wt[0:2 * H], preferred_element_type=jnp.float32)
            + jnp.dot(st, wt[2 * H:], preferred_element_type=jnp.float32)
            + bt[...])
        for g in range(G):
            ah_out[g] = ah[:, g * BLK:(g + 1) * BLK]
            at_out[g] = at[:, g * BLK:(g + 1) * BLK]


def _build_outer(a1v, a2v, kron):
    a1rep = jnp.dot(a1v, kron, preferred_element_type=jnp.float32)
    a2_128 = jnp.concatenate([a2v, a2v], axis=1)
    a2til = jnp.concatenate([a2_128] * (BLK // 2), axis=1)
    return a1rep * a2til


def _stage2_kernel(a1, a2, w, bias, kron, out_ref):
    g = pl.program_id(0)
    a1v = a1[0]
    a2v = a2[0]
    outer = _build_outer(a1v, a2v, kron[...])
    contrib = jnp.dot(outer, w[0], preferred_element_type=jnp.float32)

    @pl.when(g == 0)
    def _():
        out_ref[...] = contrib + bias[...]

    @pl.when(g > 0)
    def _():
        out_ref[...] += contrib


def kernel(hidden_state, attention, head, tail, entity_type, attention_mask,
           h_dense_w, h_dense_b, t_dense_w, t_dense_b, dis_emb, type_emb,
           cls_w, cls_b):
    f32 = jnp.float32
    g0 = jnp.asarray(_G0_NP)
    g1 = jnp.asarray(_G1_NP)
    bh = h_dense_b.reshape(1, H)
    bt = t_dense_b.reshape(1, H)

    const = lambda shape: pl.BlockSpec(shape, lambda b: tuple(0 for _ in shape))
    per_b = lambda shape: pl.BlockSpec(shape, lambda b: (b,) + tuple(0 for _ in shape[1:]))
    ah, at = pl.pallas_call(
        _stage1_kernel,
        grid=(B,),
        in_specs=[
            const((B, E)),
            const((B, E)),
            const((B, E)),
            const((B, L)),
            per_b((1, NH, L, L)),
            per_b((1, L, H)),
            const((P, E)),
            const((P, E)),
            const((HIN, H)),
            const((1, H)),
            const((HIN, H)),
            const((1, H)),
            const((TAG, TYPE)),
            const((DIS, DIS)),
        ],
        out_specs=[pl.BlockSpec((G, P, BLK), lambda b: (0, b, 0))] * 2,
        out_shape=[jax.ShapeDtypeStruct((G, B * P, BLK), f32)] * 2,
    )(head, tail, entity_type, attention_mask, attention, hidden_state, g0, g1,
      h_dense_w, bh, t_dense_w, bt, type_emb, dis_emb)

    a1 = ah
    a2 = at
    w3 = cls_w.reshape(G, BLK * BLK, REL)
    bias = cls_b.reshape(1, REL)
    kron = jnp.asarray(_KRON_NP)

    pred = pl.pallas_call(
        _stage2_kernel,
        grid=(G,),
        in_specs=[
            pl.BlockSpec((1, B * P, BLK), lambda g: (g, 0, 0)),
            pl.BlockSpec((1, B * P, BLK), lambda g: (g, 0, 0)),
            pl.BlockSpec((1, BLK * BLK, REL), lambda g: (g, 0, 0)),
            pl.BlockSpec((1, REL), lambda g: (0, 0)),
            pl.BlockSpec((BLK, BLK * BLK), lambda g: (0, 0)),
        ],
        out_specs=pl.BlockSpec((B * P, REL), lambda g: (0, 0)),
        out_shape=jax.ShapeDtypeStruct((B * P, REL), f32),
    )(a1, a2, w3, bias, kron)
    return pred


# speedup vs baseline: 1.2311x; 1.0232x over previous
from itertools import permutations

import jax
import jax.numpy as jnp
import numpy as np
from jax.experimental import pallas as pl
from jax.experimental.pallas import tpu as pltpu

B = 4
L = 512
NH = 12
H = 768
E = 16
DIS = 20
TYPE = 20
TAG = 7
REL = 97
BLK = 64
P = E * (E - 1)
G = H // BLK
HIN = H * 2 + DIS + TYPE

_HTS = np.array(list(permutations(range(E), 2)), dtype=np.int32)
_G0_NP = np.zeros((P, E), np.float32)
_G0_NP[np.arange(P), _HTS[:, 0]] = 1.0
_G1_NP = np.zeros((P, E), np.float32)
_G1_NP[np.arange(P), _HTS[:, 1]] = 1.0
_KRON_NP = np.zeros((BLK, BLK * BLK), np.float32)
for _i in range(BLK):
    _KRON_NP[_i, _i * BLK:(_i + 1) * BLK] = 1.0


def _col(row_f32):
    eye = (jax.lax.broadcasted_iota(jnp.int32, (E, E), 0)
           == jax.lax.broadcasted_iota(jnp.int32, (E, E), 1))
    diag = jnp.where(eye, jnp.broadcast_to(row_f32, (E, E)), 0.0)
    return jnp.sum(diag, axis=1, keepdims=True)


def _stage1_kernel(head_a, tail_a, et_a, mask_a, att, hs, g0, g1,
                   wh, bh, wt, bt, type_emb, dis_emb,
                   ah_out, at_out):
    b = pl.program_id(0)
    bmask = jax.lax.broadcasted_iota(jnp.int32, (B, E), 0) == b
    hrow = jnp.sum(jnp.where(bmask, head_a[...], 0), axis=0, keepdims=True)
    trow = jnp.sum(jnp.where(bmask, tail_a[...], 0), axis=0, keepdims=True)
    erow = jnp.sum(jnp.where(bmask, et_a[...], 0), axis=0, keepdims=True)
    hc_f = _col(hrow.astype(jnp.float32))
    tc_f = _col(trow.astype(jnp.float32))
    hc = hc_f.astype(jnp.int32)
    tc = tc_f.astype(jnp.int32)
    et_col = _col(erow.astype(jnp.float32)).astype(jnp.int32)
    bmask_l = jax.lax.broadcasted_iota(jnp.int32, (B, L), 0) == b
    mask = jnp.sum(jnp.where(bmask_l, mask_a[...], 0.0), axis=0, keepdims=True)

    lane_iota = jax.lax.broadcasted_iota(jnp.int32, (E, L), 1)
    s = 0.5 * ((lane_iota == hc).astype(jnp.float32)
               + (lane_iota == tc).astype(jnp.float32))
    ea = jnp.dot(s, att[0, 0], preferred_element_type=jnp.float32)
    for h in range(1, NH):
        ea += jnp.dot(s, att[0, h], preferred_element_type=jnp.float32)
    if True:
        hs2 = hs[0]
        ef = jnp.dot(s, hs2, preferred_element_type=jnp.float32)
        g0v = g0[...]
        g1v = g1[...]
        pa = (jnp.dot(g0v, ea, preferred_element_type=jnp.float32)
              * jnp.dot(g1v, ea, preferred_element_type=jnp.float32)
              * mask)
        pa = pa / (jnp.sum(pa, axis=1, keepdims=True) + 1e-20)
        info = jnp.dot(pa, hs2, preferred_element_type=jnp.float32)
        hf = jnp.dot(g0v, ef, preferred_element_type=jnp.float32)
        tf = jnp.dot(g1v, ef, preferred_element_type=jnp.float32)
        t_iota = jax.lax.broadcasted_iota(jnp.int32, (E, TAG), 1)
        t_oh = (t_iota == et_col).astype(jnp.float32)
        tfeat = jnp.dot(t_oh, type_emb[...],
                        preferred_element_type=jnp.float32)
        htype = jnp.dot(g0v, tfeat, preferred_element_type=jnp.float32)
        ttype = jnp.dot(g1v, tfeat, preferred_element_type=jnp.float32)
        x = jnp.concatenate([tc_f, hc_f], axis=1)
        y = jnp.dot(g0v, x, preferred_element_type=jnp.float32)
        z = jnp.dot(g1v, x, preferred_element_type=jnp.float32)
        d = jnp.abs(y[:, 0:1] - z[:, 1:2])
        bucket = jnp.zeros_like(d)
        for thr in (2., 4., 8., 16., 32., 64., 128., 256., 512.):
            bucket += (d >= thr).astype(jnp.float32)
        d_iota = jax.lax.broadcasted_iota(jnp.int32, (P, DIS), 1)
        d_oh = (d_iota == bucket.astype(jnp.int32)).astype(jnp.float32)
        dfeat = jnp.dot(d_oh, dis_emb[...],
                        preferred_element_type=jnp.float32)
        lh = jnp.concatenate([hf, info], axis=1)
        lt = jnp.concatenate([tf, info], axis=1)
        sh = jnp.concatenate([htype, dfeat], axis=1)
        st = jnp.concatenate([ttype, dfeat], axis=1)
        ah = jnp.tanh(
            jnp.dot(lh, wh[0:2 * H], preferred_element_type=jnp.float32)
            + jnp.dot(sh, wh[2 * H:], preferred_element_type=jnp.float32)
            + bh[...][None, :])
        at = jnp.tanh(
            jnp.dot(lt, wt[0:2 * H], preferred_element_type=jnp.float32)
            + jnp.dot(st, wt[2 * H:], preferred_element_type=jnp.float32)
            + bt[...][None, :])
        for g in range(G):
            ah_out[g] = ah[:, g * BLK:(g + 1) * BLK]
            at_out[g] = at[:, g * BLK:(g + 1) * BLK]


def _build_outer(a1v, a2v, kron):
    a1rep = jnp.dot(a1v, kron, preferred_element_type=jnp.float32)
    a2_128 = jnp.concatenate([a2v, a2v], axis=1)
    a2til = jnp.concatenate([a2_128] * (BLK // 2), axis=1)
    return a1rep * a2til


def _stage2_kernel(a1, a2, w, bias, kron, out_ref):
    g = pl.program_id(0)
    a1v = a1[0]
    a2v = a2[0]
    outer = _build_outer(a1v, a2v, kron[...])
    contrib = jnp.dot(outer, w[0], preferred_element_type=jnp.float32)

    @pl.when(g == 0)
    def _():
        out_ref[...] = contrib + bias[...][None, :]

    @pl.when(g > 0)
    def _():
        out_ref[...] += contrib


def kernel(hidden_state, attention, head, tail, entity_type, attention_mask,
           h_dense_w, h_dense_b, t_dense_w, t_dense_b, dis_emb, type_emb,
           cls_w, cls_b):
    f32 = jnp.float32
    g0 = jnp.asarray(_G0_NP)
    g1 = jnp.asarray(_G1_NP)

    const = lambda shape: pl.BlockSpec(shape, lambda b: tuple(0 for _ in shape))
    per_b = lambda shape: pl.BlockSpec(shape, lambda b: (b,) + tuple(0 for _ in shape[1:]))
    ah, at = pl.pallas_call(
        _stage1_kernel,
        grid=(B,),
        in_specs=[
            const((B, E)),
            const((B, E)),
            const((B, E)),
            const((B, L)),
            per_b((1, NH, L, L)),
            per_b((1, L, H)),
            const((P, E)),
            const((P, E)),
            const((HIN, H)),
            const((H,)),
            const((HIN, H)),
            const((H,)),
            const((TAG, TYPE)),
            const((DIS, DIS)),
        ],
        out_specs=[pl.BlockSpec((G, P, BLK), lambda b: (0, b, 0))] * 2,
        out_shape=[jax.ShapeDtypeStruct((G, B * P, BLK), f32)] * 2,
    )(head, tail, entity_type, attention_mask, attention, hidden_state, g0, g1,
      h_dense_w, h_dense_b, t_dense_w, t_dense_b, type_emb, dis_emb)

    a1 = ah
    a2 = at
    w3 = cls_w.reshape(G, BLK * BLK, REL)
    kron = jnp.asarray(_KRON_NP)

    pred = pl.pallas_call(
        _stage2_kernel,
        grid=(G,),
        in_specs=[
            pl.BlockSpec((1, B * P, BLK), lambda g: (g, 0, 0)),
            pl.BlockSpec((1, B * P, BLK), lambda g: (g, 0, 0)),
            pl.BlockSpec((1, BLK * BLK, REL), lambda g: (g, 0, 0)),
            pl.BlockSpec((REL,), lambda g: (0,)),
            pl.BlockSpec((BLK, BLK * BLK), lambda g: (0, 0)),
        ],
        out_specs=pl.BlockSpec((B * P, REL), lambda g: (0, 0)),
        out_shape=jax.ShapeDtypeStruct((B * P, REL), f32),
    )(a1, a2, w3, cls_b, kron)
    return pred


# speedup vs baseline: 1.2964x; 1.0530x over previous
from itertools import permutations

import jax
import jax.numpy as jnp
import numpy as np
from jax.experimental import pallas as pl
from jax.experimental.pallas import tpu as pltpu

B = 4
L = 512
NH = 12
H = 768
E = 16
DIS = 20
TYPE = 20
TAG = 7
REL = 97
BLK = 64
P = E * (E - 1)
G = H // BLK
HIN = H * 2 + DIS + TYPE

_HTS = np.array(list(permutations(range(E), 2)), dtype=np.int32)
_G0_NP = np.zeros((P, E), np.float32)
_G0_NP[np.arange(P), _HTS[:, 0]] = 1.0
_G1_NP = np.zeros((P, E), np.float32)
_G1_NP[np.arange(P), _HTS[:, 1]] = 1.0
_KRON_NP = np.zeros((BLK, BLK * BLK), np.float32)
for _i in range(BLK):
    _KRON_NP[_i, _i * BLK:(_i + 1) * BLK] = 1.0


def _col(row_f32):
    eye = (jax.lax.broadcasted_iota(jnp.int32, (E, E), 0)
           == jax.lax.broadcasted_iota(jnp.int32, (E, E), 1))
    diag = jnp.where(eye, jnp.broadcast_to(row_f32, (E, E)), 0.0)
    return jnp.sum(diag, axis=1, keepdims=True)


def _stage1_kernel(head_a, tail_a, et_a, mask_a, att, hs, g0, g1,
                   wh, bh, wt, bt, type_emb, dis_emb,
                   ah_out, at_out):
    b = pl.program_id(0)
    bmask = jax.lax.broadcasted_iota(jnp.int32, (B, E), 0) == b
    hrow = jnp.sum(jnp.where(bmask, head_a[...], 0), axis=0, keepdims=True)
    trow = jnp.sum(jnp.where(bmask, tail_a[...], 0), axis=0, keepdims=True)
    erow = jnp.sum(jnp.where(bmask, et_a[...], 0), axis=0, keepdims=True)
    hc_f = _col(hrow.astype(jnp.float32))
    tc_f = _col(trow.astype(jnp.float32))
    hc = hc_f.astype(jnp.int32)
    tc = tc_f.astype(jnp.int32)
    et_col = _col(erow.astype(jnp.float32)).astype(jnp.int32)
    bmask_l = jax.lax.broadcasted_iota(jnp.int32, (B, L), 0) == b
    mask = jnp.sum(jnp.where(bmask_l, mask_a[...], 0.0), axis=0, keepdims=True)

    lane_iota = jax.lax.broadcasted_iota(jnp.int32, (E, L), 1)
    s = 0.5 * ((lane_iota == hc).astype(jnp.float32)
               + (lane_iota == tc).astype(jnp.float32))
    ea = jnp.dot(s, att[0, 0], preferred_element_type=jnp.float32)
    for h in range(1, NH):
        ea += jnp.dot(s, att[0, h], preferred_element_type=jnp.float32)
    if True:
        hs2 = hs[0]
        ef = jnp.dot(s, hs2, preferred_element_type=jnp.float32)
        g0v = g0[...]
        g1v = g1[...]
        pa = (jnp.dot(g0v, ea, preferred_element_type=jnp.float32)
              * jnp.dot(g1v, ea, preferred_element_type=jnp.float32)
              * mask)
        pa = pa / (jnp.sum(pa, axis=1, keepdims=True) + 1e-20)
        info = jnp.dot(pa, hs2, preferred_element_type=jnp.float32)
        hf = jnp.dot(g0v, ef, preferred_element_type=jnp.float32)
        tf = jnp.dot(g1v, ef, preferred_element_type=jnp.float32)
        t_iota = jax.lax.broadcasted_iota(jnp.int32, (E, TAG), 1)
        t_oh = (t_iota == et_col).astype(jnp.float32)
        tfeat = jnp.dot(t_oh, type_emb[...],
                        preferred_element_type=jnp.float32)
        htype = jnp.dot(g0v, tfeat, preferred_element_type=jnp.float32)
        ttype = jnp.dot(g1v, tfeat, preferred_element_type=jnp.float32)
        x = jnp.concatenate([tc_f, hc_f], axis=1)
        y = jnp.dot(g0v, x, preferred_element_type=jnp.float32)
        z = jnp.dot(g1v, x, preferred_element_type=jnp.float32)
        d = jnp.abs(y[:, 0:1] - z[:, 1:2])
        bucket = jnp.zeros_like(d)
        for thr in (2., 4., 8., 16., 32., 64., 128., 256., 512.):
            bucket += (d >= thr).astype(jnp.float32)
        d_iota = jax.lax.broadcasted_iota(jnp.int32, (P, DIS), 1)
        d_oh = (d_iota == bucket.astype(jnp.int32)).astype(jnp.float32)
        dfeat = jnp.dot(d_oh, dis_emb[...],
                        preferred_element_type=jnp.float32)
        lh = jnp.concatenate([hf, info], axis=1)
        lt = jnp.concatenate([tf, info], axis=1)
        sh = jnp.concatenate([htype, dfeat], axis=1)
        st = jnp.concatenate([ttype, dfeat], axis=1)
        ah = jnp.tanh(
            jnp.dot(lh, wh[0:2 * H], preferred_element_type=jnp.float32)
            + jnp.dot(sh, wh[2 * H:], preferred_element_type=jnp.float32)
            + bh[...][None, :])
        at = jnp.tanh(
            jnp.dot(lt, wt[0:2 * H], preferred_element_type=jnp.float32)
            + jnp.dot(st, wt[2 * H:], preferred_element_type=jnp.float32)
            + bt[...][None, :])
        for g in range(G):
            ah_out[g] = ah[:, g * BLK:(g + 1) * BLK]
            at_out[g] = at[:, g * BLK:(g + 1) * BLK]


def _build_outer(a1v, a2v, kron):
    a1rep = jnp.dot(a1v, kron, preferred_element_type=jnp.float32)
    a2_128 = jnp.concatenate([a2v, a2v], axis=1)
    a2til = jnp.concatenate([a2_128] * (BLK // 2), axis=1)
    return a1rep * a2til


def _stage2_kernel(a1, a2, w, bias, kron, out_ref):
    g = pl.program_id(0)
    a1v = a1[0]
    a2v = a2[0]
    outer = _build_outer(a1v, a2v, kron[...])
    contrib = jnp.dot(outer, w[...], preferred_element_type=jnp.float32)

    @pl.when(g == 0)
    def _():
        out_ref[...] = contrib + bias[...][None, :]

    @pl.when(g > 0)
    def _():
        out_ref[...] += contrib


def kernel(hidden_state, attention, head, tail, entity_type, attention_mask,
           h_dense_w, h_dense_b, t_dense_w, t_dense_b, dis_emb, type_emb,
           cls_w, cls_b):
    f32 = jnp.float32
    g0 = jnp.asarray(_G0_NP)
    g1 = jnp.asarray(_G1_NP)

    const = lambda shape: pl.BlockSpec(shape, lambda b: tuple(0 for _ in shape))
    per_b = lambda shape: pl.BlockSpec(shape, lambda b: (b,) + tuple(0 for _ in shape[1:]))
    ah, at = pl.pallas_call(
        _stage1_kernel,
        grid=(B,),
        in_specs=[
            const((B, E)),
            const((B, E)),
            const((B, E)),
            const((B, L)),
            per_b((1, NH, L, L)),
            per_b((1, L, H)),
            const((P, E)),
            const((P, E)),
            const((HIN, H)),
            const((H,)),
            const((HIN, H)),
            const((H,)),
            const((TAG, TYPE)),
            const((DIS, DIS)),
        ],
        out_specs=[pl.BlockSpec((G, P, BLK), lambda b: (0, b, 0))] * 2,
        out_shape=[jax.ShapeDtypeStruct((G, B * P, BLK), f32)] * 2,
    )(head, tail, entity_type, attention_mask, attention, hidden_state, g0, g1,
      h_dense_w, h_dense_b, t_dense_w, t_dense_b, type_emb, dis_emb)

    a1 = ah
    a2 = at
    kron = jnp.asarray(_KRON_NP)

    pred = pl.pallas_call(
        _stage2_kernel,
        grid=(G,),
        in_specs=[
            pl.BlockSpec((1, B * P, BLK), lambda g: (g, 0, 0)),
            pl.BlockSpec((1, B * P, BLK), lambda g: (g, 0, 0)),
            pl.BlockSpec((BLK * BLK, REL), lambda g: (g, 0)),
            pl.BlockSpec((REL,), lambda g: (0,)),
            pl.BlockSpec((BLK, BLK * BLK), lambda g: (0, 0)),
        ],
        out_specs=pl.BlockSpec((B * P, REL), lambda g: (0, 0)),
        out_shape=jax.ShapeDtypeStruct((B * P, REL), f32),
    )(a1, a2, cls_w, cls_b, kron)
    return pred
